# Initial kernel scaffold; baseline (speedup 1.0000x reference)
import functools

import jax
import jax.numpy as jnp
from jax import lax
from jax.experimental import pallas as pl
from jax.experimental.pallas import tpu as pltpu
from jax.experimental.pallas import tpu_sc as plsc

RETRIEVE_K = 20
_NEG_INF = float("-inf")
_INT_MAX = 2**31 - 1


def _user_rep_body(seq_ref, w_ref, b_ref, out_ref):
    inv_l = 1.0 / seq_ref.shape[1]
    m = jnp.sum(seq_ref[...], axis=1) * inv_l
    mm = lax.dot_general(m, w_ref[...], (((1,), (0,)), ((), ())),
                         preferred_element_type=jnp.float32)
    out_ref[...] = jnp.tanh(mm + b_ref[...][None, :])


def _user_rep_call(seq, w_llm, b_llm):
    B, L, D = seq.shape
    bb = 256
    return pl.pallas_call(
        _user_rep_body,
        grid=(B // bb,),
        in_specs=[
            pl.BlockSpec((bb, L, D), lambda b: (b, 0, 0)),
            pl.BlockSpec((D, D), lambda b: (0, 0)),
            pl.BlockSpec((D,), lambda b: (0,)),
        ],
        out_specs=pl.BlockSpec((bb, D), lambda b: (b, 0)),
        out_shape=jax.ShapeDtypeStruct((B, D), jnp.float32),
    )(seq, w_llm, b_llm)


_R_PAD = 128


def _topk_body(urep_ref, item_ref, oidx_ref, rv_ref, ri_ref, *, v_total, k):
    v = pl.program_id(1)
    nv = pl.num_programs(1)
    bb = urep_ref.shape[0]
    vb = item_ref.shape[0]

    @pl.when(v == 0)
    def _init():
        rv_ref[...] = jnp.full(rv_ref.shape, _NEG_INF, jnp.float32)
        ri_ref[...] = jnp.zeros(ri_ref.shape, jnp.int32)

    s = lax.dot_general(urep_ref[...], item_ref[...], (((1,), (1,)), ((), ())),
                        preferred_element_type=jnp.float32)
    col = lax.broadcasted_iota(jnp.int32, (bb, vb), 1) + v * vb
    s = jnp.where(col < v_total, s, _NEG_INF)

    vals = jnp.concatenate([s, rv_ref[...]], axis=1)
    ids = jnp.concatenate([col, ri_ref[...]], axis=1)
    top_v, top_i = [], []
    for _ in range(k):
        m = jnp.max(vals, axis=1, keepdims=True)
        am = jnp.min(jnp.where(vals == m, ids, _INT_MAX), axis=1, keepdims=True)
        top_v.append(m)
        top_i.append(am)
        vals = jnp.where(ids == am, _NEG_INF, vals)

    rv_ref[...] = jnp.concatenate(
        top_v + [jnp.full((bb, _R_PAD - k), _NEG_INF, jnp.float32)], axis=1)
    ri_ref[...] = jnp.concatenate(
        top_i + [jnp.zeros((bb, _R_PAD - k), jnp.int32)], axis=1)

    @pl.when(v == nv - 1)
    def _emit():
        oidx_ref[...] = jnp.concatenate(top_i, axis=1)


def _topk_call(urep, item, k):
    B, D = urep.shape
    V = item.shape[0]
    bb = 256
    vb = 2048
    nv = pl.cdiv(V, vb)
    return pl.pallas_call(
        functools.partial(_topk_body, v_total=V, k=k),
        grid=(B // bb, nv),
        in_specs=[
            pl.BlockSpec((bb, D), lambda b, v: (b, 0)),
            pl.BlockSpec((vb, D), lambda b, v: (v, 0)),
        ],
        out_specs=pl.BlockSpec((bb, k), lambda b, v: (b, 0)),
        out_shape=jax.ShapeDtypeStruct((B, k), jnp.int32),
        scratch_shapes=[
            pltpu.VMEM((bb, _R_PAD), jnp.float32),
            pltpu.VMEM((bb, _R_PAD), jnp.int32),
        ],
        compiler_params=pltpu.CompilerParams(
            dimension_semantics=("arbitrary", "arbitrary")),
    )(urep, item)


def _make_sc_gather_mean(V, D, B, k):
    info = plsc.get_sparse_core_info()
    nw = info.num_cores * info.num_subcores
    bpw = B // nw
    nlane = info.num_lanes
    nch = D // nlane
    mesh = plsc.VectorSubcoreMesh(core_axis_name="c", subcore_axis_name="s")

    @functools.partial(
        pl.kernel,
        out_type=jax.ShapeDtypeStruct((B, D), jnp.float32),
        mesh=mesh,
        scratch_types=[
            pltpu.VMEM((bpw, k), jnp.int32),
            pltpu.VMEM((k, D), jnp.float32),
            pltpu.VMEM((1, D), jnp.float32),
            pltpu.SemaphoreType.DMA,
        ],
    )
    def sc_gather_mean(table_hbm, idx_hbm, out_hbm, idx_v, rows_v, stage_v, sem):
        wid = lax.axis_index("s") * info.num_cores + lax.axis_index("c")
        pltpu.sync_copy(idx_hbm.at[wid], idx_v)

        def body(bi, carry):
            pltpu.async_copy(table_hbm.at[idx_v.at[bi]], rows_v, sem).wait()
            for c in range(nch):
                sl = pl.ds(c * nlane, nlane)
                acc = rows_v[0, sl]
                for r in range(1, k):
                    acc = acc + rows_v[r, sl]
                stage_v[0, sl] = acc * (1.0 / k)
            pltpu.sync_copy(stage_v, out_hbm.at[pl.ds(wid * bpw + bi, 1)])
            return carry

        lax.fori_loop(0, bpw, body, 0)

    return sc_gather_mean, nw, bpw


def _gather_mean_call(item, idx):
    V, D = item.shape
    B, k = idx.shape
    fn, nw, bpw = _make_sc_gather_mean(V, D, B, k)
    return fn(item, idx.reshape(nw, bpw, k))


def _proj_body(urep_ref, retr_ref, wf_ref, bf_ref, wp_ref, bp_ref, out_ref,
               fused_ref):
    D = urep_ref.shape[1]

    @pl.when(pl.program_id(0) == 0)
    def _fuse():
        f = lax.dot_general(urep_ref[...], wf_ref[0:D, :],
                            (((1,), (0,)), ((), ())),
                            preferred_element_type=jnp.float32)
        f = f + lax.dot_general(retr_ref[...], wf_ref[D:2 * D, :],
                                (((1,), (0,)), ((), ())),
                                preferred_element_type=jnp.float32)
        fused_ref[...] = f + bf_ref[...][None, :]

    out_ref[...] = lax.dot_general(fused_ref[...], wp_ref[...],
                                   (((1,), (0,)), ((), ())),
                                   preferred_element_type=jnp.float32
                                   ) + bp_ref[...][None, :]


def _proj_call(urep, retr, w_fusion, b_fusion, w_proj, b_proj):
    B, D = urep.shape
    V = w_proj.shape[1]
    vb = 2048
    nv = pl.cdiv(V, vb)
    return pl.pallas_call(
        _proj_body,
        grid=(nv,),
        in_specs=[
            pl.BlockSpec((B, D), lambda v: (0, 0)),
            pl.BlockSpec((B, D), lambda v: (0, 0)),
            pl.BlockSpec((2 * D, D), lambda v: (0, 0)),
            pl.BlockSpec((D,), lambda v: (0,)),
            pl.BlockSpec((D, vb), lambda v: (0, v)),
            pl.BlockSpec((vb,), lambda v: (v,)),
        ],
        out_specs=pl.BlockSpec((B, vb), lambda v: (0, v)),
        out_shape=jax.ShapeDtypeStruct((B, V), jnp.float32),
        scratch_shapes=[pltpu.VMEM((B, D), jnp.float32)],
        compiler_params=pltpu.CompilerParams(
            dimension_semantics=("arbitrary",)),
    )(urep, retr, w_fusion, b_fusion, w_proj, b_proj)


def kernel(sequence_embeddings, W_llm, b_llm, item_embeddings, W_fusion,
           b_fusion, W_proj, b_proj):
    user_rep = _user_rep_call(sequence_embeddings, W_llm, b_llm)
    idx = _topk_call(user_rep, item_embeddings, RETRIEVE_K)
    retrieved = _gather_mean_call(item_embeddings, idx)
    return _proj_call(user_rep, retrieved, W_fusion, b_fusion, W_proj, b_proj)


# speedup vs baseline: 1.2834x; 1.2834x over previous
import functools

import jax
import jax.numpy as jnp
from jax import lax
from jax.experimental import pallas as pl
from jax.experimental.pallas import tpu as pltpu
from jax.experimental.pallas import tpu_sc as plsc

RETRIEVE_K = 20
_NEG_INF = float("-inf")
_INT_MAX = 2**31 - 1


def _user_rep_body(seq_ref, w_ref, b_ref, out_ref):
    L = seq_ref.shape[1]
    inv_l = 1.0 / L
    m = seq_ref[:, 0, :]
    for r in range(1, L):
        m = m + seq_ref[:, r, :]
    m = m * inv_l
    mm = lax.dot_general(m, w_ref[...], (((1,), (0,)), ((), ())),
                         preferred_element_type=jnp.float32)
    out_ref[...] = mm + b_ref[...][None, :]


def _user_rep_call(seq, w_llm, b_llm):
    B, L, D = seq.shape
    bb = 128
    return pl.pallas_call(
        _user_rep_body,
        grid=(B // bb,),
        in_specs=[
            pl.BlockSpec((bb, L, D), lambda b: (b, 0, 0)),
            pl.BlockSpec((D, D), lambda b: (0, 0)),
            pl.BlockSpec((D,), lambda b: (0,)),
        ],
        out_specs=pl.BlockSpec((bb, D), lambda b: (b, 0)),
        out_shape=jax.ShapeDtypeStruct((B, D), jnp.float32),
    )(seq, w_llm, b_llm)


def _topk_part_body(urep_ref, item_ref, ov_ref, oi_ref, *, v_total, k):
    v = pl.program_id(1)
    bb = urep_ref.shape[0]
    vb = item_ref.shape[0]

    s = lax.dot_general(urep_ref[...], item_ref[...], (((1,), (1,)), ((), ())),
                        preferred_element_type=jnp.float32)
    col = lax.broadcasted_iota(jnp.int32, (bb, vb), 1) + v * vb
    s = jnp.where(col < v_total, s, _NEG_INF)

    top_v, top_i = [], []
    for _ in range(k):
        m = jnp.max(s, axis=1, keepdims=True)
        am = jnp.min(jnp.where(s == m, col, _INT_MAX), axis=1, keepdims=True)
        top_v.append(m)
        top_i.append(am)
        s = jnp.where(col == am, _NEG_INF, s)

    ov_ref[...] = jnp.concatenate(top_v, axis=1)[None]
    oi_ref[...] = jnp.concatenate(top_i, axis=1)[None]


def _topk_merge_body(cv_ref, ci_ref, oidx_ref, *, k):
    vals = cv_ref[...]
    ids = ci_ref[...]
    top_i = []
    for _ in range(k):
        m = jnp.max(jnp.max(vals, axis=2), axis=0)
        mb = m[None, :, None]
        am3 = jnp.where(vals == mb, ids, _INT_MAX)
        am = jnp.min(jnp.min(am3, axis=2), axis=0)
        top_i.append(am[:, None])
        vals = jnp.where(ids == am[None, :, None], _NEG_INF, vals)
    oidx_ref[...] = jnp.concatenate(top_i, axis=1)


def _topk_call(urep, item, k):
    B, D = urep.shape
    V = item.shape[0]
    bb = 256
    vb = 2048
    nv = pl.cdiv(V, vb)
    cand_v, cand_i = pl.pallas_call(
        functools.partial(_topk_part_body, v_total=V, k=k),
        grid=(B // bb, nv),
        in_specs=[
            pl.BlockSpec((bb, D), lambda b, v: (b, 0)),
            pl.BlockSpec((vb, D), lambda b, v: (v, 0)),
        ],
        out_specs=[
            pl.BlockSpec((1, bb, k), lambda b, v: (v, b, 0)),
            pl.BlockSpec((1, bb, k), lambda b, v: (v, b, 0)),
        ],
        out_shape=[
            jax.ShapeDtypeStruct((nv, B, k), jnp.float32),
            jax.ShapeDtypeStruct((nv, B, k), jnp.int32),
        ],
        compiler_params=pltpu.CompilerParams(
            dimension_semantics=("parallel", "arbitrary")),
    )(urep, item)

    return pl.pallas_call(
        functools.partial(_topk_merge_body, k=k),
        grid=(B // bb,),
        in_specs=[
            pl.BlockSpec((nv, bb, k), lambda b: (0, b, 0)),
            pl.BlockSpec((nv, bb, k), lambda b: (0, b, 0)),
        ],
        out_specs=pl.BlockSpec((bb, k), lambda b: (b, 0)),
        out_shape=jax.ShapeDtypeStruct((B, k), jnp.int32),
    )(cand_v, cand_i)


_SC_W = 128


def _make_sc_gather_mean(V, D, B, k):
    info = plsc.get_sparse_core_info()
    nw = info.num_cores * info.num_subcores
    bpw = B // nw
    nlane = info.num_lanes
    nsplit = D // _SC_W
    rows_per_b = k * nsplit
    mesh = plsc.VectorSubcoreMesh(core_axis_name="c", subcore_axis_name="s")

    @functools.partial(
        pl.kernel,
        out_type=jax.ShapeDtypeStruct((B, D), jnp.float32),
        mesh=mesh,
        scratch_types=[
            pltpu.VMEM((rows_per_b,), jnp.int32),
            pltpu.VMEM((rows_per_b, _SC_W), jnp.float32),
            pltpu.VMEM((1, D), jnp.float32),
            pltpu.SemaphoreType.DMA,
        ],
    )
    def sc_gather_mean(table_hbm, idx_hbm, out_hbm, idx_v, rows_v, stage_v, sem):
        wid = lax.axis_index("s") * info.num_cores + lax.axis_index("c")

        def body(bi, carry):
            pltpu.sync_copy(idx_hbm.at[wid, bi], idx_v)
            pltpu.async_copy(table_hbm.at[idx_v], rows_v, sem).wait()
            for c in range(nsplit):
                for sub in range(_SC_W // nlane):
                    sl_in = pl.ds(sub * nlane, nlane)
                    acc = rows_v[c, sl_in]
                    for r in range(1, k):
                        acc = acc + rows_v[r * nsplit + c, sl_in]
                    stage_v[0, pl.ds(c * _SC_W + sub * nlane, nlane)] = (
                        acc * (1.0 / k))
            pltpu.sync_copy(stage_v, out_hbm.at[pl.ds(wid * bpw + bi, 1)])
            return carry

        lax.fori_loop(0, bpw, body, 0)

    return sc_gather_mean, nw, bpw


def _gather_mean_call(item, idx):
    V, D = item.shape
    B, k = idx.shape
    fn, nw, bpw = _make_sc_gather_mean(V, D, B, k)
    nsplit = D // _SC_W
    table2 = item.reshape(V * nsplit, _SC_W)
    idx4 = (idx[:, :, None] * nsplit
            + jnp.arange(nsplit, dtype=jnp.int32)[None, None, :])
    return fn(table2, idx4.reshape(nw, bpw, k * nsplit))


def _fusion_body(urep_ref, retr_ref, wf_ref, bf_ref, out_ref):
    D = urep_ref.shape[1]
    f = lax.dot_general(urep_ref[...], wf_ref[0:D, :],
                        (((1,), (0,)), ((), ())),
                        preferred_element_type=jnp.float32)
    f = f + lax.dot_general(retr_ref[...], wf_ref[D:2 * D, :],
                            (((1,), (0,)), ((), ())),
                            preferred_element_type=jnp.float32)
    out_ref[...] = f + bf_ref[...][None, :]


def _fusion_call(urep, retr, w_fusion, b_fusion):
    B, D = urep.shape
    return pl.pallas_call(
        _fusion_body,
        out_shape=jax.ShapeDtypeStruct((B, D), jnp.float32),
    )(urep, retr, w_fusion, b_fusion)


def _proj_body(fused_ref, wp_ref, bp_ref, out_ref):
    out_ref[...] = lax.dot_general(fused_ref[...], wp_ref[...],
                                   (((1,), (0,)), ((), ())),
                                   preferred_element_type=jnp.float32
                                   ) + bp_ref[...][None, :]


def _proj_call(fused, w_proj, b_proj):
    B, D = fused.shape
    V = w_proj.shape[1]
    vb = 2048
    nv = pl.cdiv(V, vb)
    return pl.pallas_call(
        _proj_body,
        grid=(nv,),
        in_specs=[
            pl.BlockSpec((B, D), lambda v: (0, 0)),
            pl.BlockSpec((D, vb), lambda v: (0, v)),
            pl.BlockSpec((vb,), lambda v: (v,)),
        ],
        out_specs=pl.BlockSpec((B, vb), lambda v: (0, v)),
        out_shape=jax.ShapeDtypeStruct((B, V), jnp.float32),
        compiler_params=pltpu.CompilerParams(
            dimension_semantics=("arbitrary",)),
    )(fused, w_proj, b_proj)


def kernel(sequence_embeddings, W_llm, b_llm, item_embeddings, W_fusion,
           b_fusion, W_proj, b_proj):
    user_rep = jnp.tanh(_user_rep_call(sequence_embeddings, W_llm, b_llm))
    idx = _topk_call(user_rep, item_embeddings, RETRIEVE_K)
    retrieved = _gather_mean_call(item_embeddings, idx)
    fused = _fusion_call(user_rep, retrieved, W_fusion, b_fusion)
    return _proj_call(fused, W_proj, b_proj)


# speedup vs baseline: 1.4906x; 1.1615x over previous
import functools

import jax
import jax.numpy as jnp
from jax import lax
from jax.experimental import pallas as pl
from jax.experimental.pallas import tpu as pltpu
from jax.experimental.pallas import tpu_sc as plsc

RETRIEVE_K = 20
_NEG_INF = float("-inf")
_INT_MAX = 2**31 - 1


def _user_rep_body(seq_ref, w_ref, b_ref, out_ref):
    L = seq_ref.shape[1]
    inv_l = 1.0 / L
    m = seq_ref[:, 0, :]
    for r in range(1, L):
        m = m + seq_ref[:, r, :]
    m = m * inv_l
    mm = lax.dot_general(m, w_ref[...], (((1,), (0,)), ((), ())),
                         preferred_element_type=jnp.float32)
    out_ref[...] = mm + b_ref[...][None, :]


def _user_rep_call(seq, w_llm, b_llm):
    B, L, D = seq.shape
    bb = 128
    return pl.pallas_call(
        _user_rep_body,
        grid=(B // bb,),
        in_specs=[
            pl.BlockSpec((bb, L, D), lambda b: (b, 0, 0)),
            pl.BlockSpec((D, D), lambda b: (0, 0)),
            pl.BlockSpec((D,), lambda b: (0,)),
        ],
        out_specs=pl.BlockSpec((bb, D), lambda b: (b, 0)),
        out_shape=jax.ShapeDtypeStruct((B, D), jnp.float32),
    )(seq, w_llm, b_llm)


_CPAD = 128


def _topk_part_body(urep_ref, item_ref, ov_ref, oi_ref, *, v_total, k):
    v = pl.program_id(1)
    bb = urep_ref.shape[0]
    vb = item_ref.shape[0]

    s = lax.dot_general(urep_ref[...], item_ref[...], (((1,), (1,)), ((), ())),
                        preferred_element_type=jnp.float32)
    col = lax.broadcasted_iota(jnp.int32, (bb, vb), 1) + v * vb
    s = jnp.where(col < v_total, s, _NEG_INF)

    top_v, top_i = [], []
    for _ in range(k):
        m = jnp.max(s, axis=1, keepdims=True)
        am = jnp.min(jnp.where(s == m, col, _INT_MAX), axis=1, keepdims=True)
        top_v.append(m)
        top_i.append(am)
        s = jnp.where(col == am, _NEG_INF, s)

    ov_ref[...] = jnp.concatenate(
        top_v + [jnp.full((bb, _CPAD - k), _NEG_INF, jnp.float32)], axis=1)
    oi_ref[...] = jnp.concatenate(
        top_i + [jnp.zeros((bb, _CPAD - k), jnp.int32)], axis=1)


def _topk_merge_body(cv_ref, ci_ref, oidx_ref, *, k):
    vals = cv_ref[...]
    ids = ci_ref[...]
    top_i = []
    for _ in range(k):
        m = jnp.max(vals, axis=1, keepdims=True)
        am = jnp.min(jnp.where(vals == m, ids, _INT_MAX), axis=1, keepdims=True)
        top_i.append(am)
        vals = jnp.where(ids == am, _NEG_INF, vals)
    oidx_ref[...] = jnp.concatenate(top_i, axis=1)


def _topk_call(urep, item, k):
    B, D = urep.shape
    V = item.shape[0]
    bb = 256
    vb = 4096
    nv = pl.cdiv(V, vb)
    cand_v, cand_i = pl.pallas_call(
        functools.partial(_topk_part_body, v_total=V, k=k),
        grid=(B // bb, nv),
        in_specs=[
            pl.BlockSpec((bb, D), lambda b, v: (b, 0)),
            pl.BlockSpec((vb, D), lambda b, v: (v, 0)),
        ],
        out_specs=[
            pl.BlockSpec((bb, _CPAD), lambda b, v: (b, v)),
            pl.BlockSpec((bb, _CPAD), lambda b, v: (b, v)),
        ],
        out_shape=[
            jax.ShapeDtypeStruct((B, nv * _CPAD), jnp.float32),
            jax.ShapeDtypeStruct((B, nv * _CPAD), jnp.int32),
        ],
        compiler_params=pltpu.CompilerParams(
            dimension_semantics=("parallel", "arbitrary")),
    )(urep, item)

    return pl.pallas_call(
        functools.partial(_topk_merge_body, k=k),
        grid=(B // bb,),
        in_specs=[
            pl.BlockSpec((bb, nv * _CPAD), lambda b: (b, 0)),
            pl.BlockSpec((bb, nv * _CPAD), lambda b: (b, 0)),
        ],
        out_specs=pl.BlockSpec((bb, k), lambda b: (b, 0)),
        out_shape=jax.ShapeDtypeStruct((B, k), jnp.int32),
    )(cand_v, cand_i)


_SC_W = 128


def _make_sc_gather_mean(V, D, B, k):
    info = plsc.get_sparse_core_info()
    nw = info.num_cores * info.num_subcores
    bpw = B // nw
    nlane = info.num_lanes
    nsplit = D // _SC_W
    rows_per_b = k * nsplit
    mesh = plsc.VectorSubcoreMesh(core_axis_name="c", subcore_axis_name="s")

    @functools.partial(
        pl.kernel,
        out_type=jax.ShapeDtypeStruct((B, D), jnp.float32),
        mesh=mesh,
        scratch_types=[
            pltpu.VMEM((rows_per_b,), jnp.int32),
            pltpu.VMEM((rows_per_b, _SC_W), jnp.float32),
            pltpu.VMEM((1, D), jnp.float32),
            pltpu.SemaphoreType.DMA,
        ],
    )
    def sc_gather_mean(table_hbm, idx_hbm, out_hbm, idx_v, rows_v, stage_v, sem):
        wid = lax.axis_index("s") * info.num_cores + lax.axis_index("c")

        def body(bi, carry):
            pltpu.sync_copy(idx_hbm.at[wid, bi], idx_v)
            pltpu.async_copy(table_hbm.at[idx_v], rows_v, sem).wait()
            for c in range(nsplit):
                for sub in range(_SC_W // nlane):
                    sl_in = pl.ds(sub * nlane, nlane)
                    acc = rows_v[c, sl_in]
                    for r in range(1, k):
                        acc = acc + rows_v[r * nsplit + c, sl_in]
                    stage_v[0, pl.ds(c * _SC_W + sub * nlane, nlane)] = (
                        acc * (1.0 / k))
            pltpu.sync_copy(stage_v, out_hbm.at[pl.ds(wid * bpw + bi, 1)])
            return carry

        lax.fori_loop(0, bpw, body, 0)

    return sc_gather_mean, nw, bpw


def _gather_mean_call(item, idx):
    V, D = item.shape
    B, k = idx.shape
    fn, nw, bpw = _make_sc_gather_mean(V, D, B, k)
    nsplit = D // _SC_W
    table2 = item.reshape(V * nsplit, _SC_W)
    idx4 = (idx[:, :, None] * nsplit
            + jnp.arange(nsplit, dtype=jnp.int32)[None, None, :])
    return fn(table2, idx4.reshape(nw, bpw, k * nsplit))


def _fusion_body(urep_ref, retr_ref, wf_ref, bf_ref, out_ref):
    D = urep_ref.shape[1]
    f = lax.dot_general(urep_ref[...], wf_ref[0:D, :],
                        (((1,), (0,)), ((), ())),
                        preferred_element_type=jnp.float32)
    f = f + lax.dot_general(retr_ref[...], wf_ref[D:2 * D, :],
                            (((1,), (0,)), ((), ())),
                            preferred_element_type=jnp.float32)
    out_ref[...] = f + bf_ref[...][None, :]


def _fusion_call(urep, retr, w_fusion, b_fusion):
    B, D = urep.shape
    return pl.pallas_call(
        _fusion_body,
        out_shape=jax.ShapeDtypeStruct((B, D), jnp.float32),
    )(urep, retr, w_fusion, b_fusion)


def _proj_body(fused_ref, wp_ref, bp_ref, out_ref):
    out_ref[...] = lax.dot_general(fused_ref[...], wp_ref[...],
                                   (((1,), (0,)), ((), ())),
                                   preferred_element_type=jnp.float32
                                   ) + bp_ref[...][None, :]


def _proj_call(fused, w_proj, b_proj):
    B, D = fused.shape
    V = w_proj.shape[1]
    vb = 2048
    nv = pl.cdiv(V, vb)
    return pl.pallas_call(
        _proj_body,
        grid=(nv,),
        in_specs=[
            pl.BlockSpec((B, D), lambda v: (0, 0)),
            pl.BlockSpec((D, vb), lambda v: (0, v)),
            pl.BlockSpec((vb,), lambda v: (v,)),
        ],
        out_specs=pl.BlockSpec((B, vb), lambda v: (0, v)),
        out_shape=jax.ShapeDtypeStruct((B, V), jnp.float32),
        compiler_params=pltpu.CompilerParams(
            dimension_semantics=("arbitrary",)),
    )(fused, w_proj, b_proj)


def kernel(sequence_embeddings, W_llm, b_llm, item_embeddings, W_fusion,
           b_fusion, W_proj, b_proj):
    user_rep = jnp.tanh(_user_rep_call(sequence_embeddings, W_llm, b_llm))
    idx = _topk_call(user_rep, item_embeddings, RETRIEVE_K)
    retrieved = _gather_mean_call(item_embeddings, idx)
    fused = _fusion_call(user_rep, retrieved, W_fusion, b_fusion)
    return _proj_call(fused, W_proj, b_proj)
